# baseline (speedup 1.0000x reference)
import functools

import jax
import jax.numpy as jnp
from jax import lax
from jax.experimental import pallas as pl
from jax.experimental.pallas import tpu as pltpu
from jax.experimental.pallas import tpu_sc as plsc

ALPHA = 1.702
LIMIT = 7.0
BM = 256
RB = 128
NSC = 32


def _dot(a, b, prec=None):
    return jax.lax.dot_general(
        a, b, (((1,), (0,)), ((), ())),
        precision=prec, preferred_element_type=jnp.float32)


def _rs_kernel(x_ref, rw_ref, rb_ref, pa_ref, pb_ref, pos_ref, blk_ref,
               os_ref, carry_ref, carry2_ref, start_ref):
    ph = pl.program_id(0)
    m = pl.program_id(1)
    E = rw_ref.shape[1]
    NTB = os_ref.shape[0] // 2

    @pl.when((ph == 0) & (m < NTB))
    def _router():
        l = _dot(x_ref[...], rw_ref[...]) + rb_ref[...]
        m1 = jnp.max(l, axis=1, keepdims=True)
        tri = (jax.lax.broadcasted_iota(jnp.int32, (E, E), 0)
               <= jax.lax.broadcasted_iota(jnp.int32, (E, E), 1)).astype(jnp.float32)
        is1 = (l == m1).astype(jnp.float32)
        sel1 = is1 * (_dot(is1, tri) == 1.0).astype(jnp.float32)
        l2 = jnp.where(sel1 > 0.0, -jnp.inf, l)
        m2 = jnp.max(l2, axis=1, keepdims=True)
        is2 = (l2 == m2).astype(jnp.float32)
        sel2 = is2 * (_dot(is2, tri) == 1.0).astype(jnp.float32)
        e2 = jnp.exp(m2 - m1)
        p1 = 1.0 / (1.0 + e2)
        p2 = e2 / (1.0 + e2)
        os_ref[pl.ds(m, 1)] = sel1.reshape(1, RB, E)
        os_ref[pl.ds(m + NTB, 1)] = sel2.reshape(1, RB, E)
        pa_ref[...] = p1
        pb_ref[...] = p2

    @pl.when(ph == 0)
    def _count():
        @pl.when(m == 0)
        def _zero():
            carry_ref[...] = jnp.zeros_like(carry_ref)

        ob = os_ref[pl.ds(m, 1)].reshape(RB, E)
        carry_ref[...] = carry_ref[...] + jnp.sum(ob, axis=0, keepdims=True)

    @pl.when(ph == 1)
    def _pos():
        E_ = carry_ref.shape[1]

        @pl.when(m == 0)
        def _starts():
            carry2_ref[...] = jnp.zeros_like(carry2_ref)
            tot = carry_ref[...]
            ptot = jnp.floor((tot + (BM - 1.0)) * (1.0 / BM)) * BM
            u8s = (jax.lax.broadcasted_iota(jnp.int32, (E_, E_), 0)
                   < jax.lax.broadcasted_iota(jnp.int32, (E_, E_), 1)).astype(jnp.float32)
            start = _dot(ptot, u8s)
            start_ref[...] = start
            nlan = blk_ref.shape[1]
            rows0 = jax.lax.broadcasted_iota(
                jnp.int32, (1, nlan), 1).astype(jnp.float32) * BM
            blk = jnp.full((1, nlan), -1, jnp.int32)
            for e in range(E_):
                rep = (jax.lax.broadcasted_iota(jnp.int32, (E_, nlan), 0)
                       == e).astype(jnp.float32)
                se = _dot(start, rep)
                pe = _dot(ptot, rep)
                mm = (rows0 >= se) & (rows0 < se + pe)
                blk = jnp.where(mm, e, blk)
            blk_ref[...] = blk

        ob = os_ref[pl.ds(m, 1)].reshape(RB, E)
        ltri = (jax.lax.broadcasted_iota(jnp.int32, (RB, RB), 0)
                > jax.lax.broadcasted_iota(jnp.int32, (RB, RB), 1)).astype(jnp.float32)
        pre = _dot(ltri, ob) + carry2_ref[...]
        rank = jnp.sum(ob * pre, axis=1, keepdims=True)
        carry2_ref[...] = carry2_ref[...] + jnp.sum(ob, axis=0, keepdims=True)
        sv = jax.lax.dot_general(ob, start_ref[...], (((1,), (1,)), ((), ())),
                                 preferred_element_type=jnp.float32)
        pos_ref[...] = (sv + rank).astype(jnp.int32)


def _router_sort(x, router_w, router_b, NB):
    T, H = x.shape
    E = router_w.shape[1]
    NTB = T // RB
    NPB = 2 * NTB
    pa, pb, pos, blk = pl.pallas_call(
        _rs_kernel,
        grid=(2, NPB),
        in_specs=[
            pl.BlockSpec((RB, H), lambda ph, m: (
                jnp.where(ph == 0, jnp.minimum(m, NTB - 1), NTB - 1), 0)),
            pl.BlockSpec((H, E), lambda ph, m: (0, 0)),
            pl.BlockSpec((1, E), lambda ph, m: (0, 0)),
        ],
        out_specs=[
            pl.BlockSpec((RB, 1), lambda ph, m: (
                jnp.where(ph == 0, jnp.minimum(m, NTB - 1), NTB - 1), 0)),
            pl.BlockSpec((RB, 1), lambda ph, m: (
                jnp.where(ph == 0, jnp.minimum(m, NTB - 1), NTB - 1), 0)),
            pl.BlockSpec((RB, 1), lambda ph, m: (
                jnp.where(ph == 1, m, NPB), 0)),
            pl.BlockSpec((1, 32), lambda ph, m: (0, 0)),
        ],
        out_shape=[
            jax.ShapeDtypeStruct((NTB * RB, 1), jnp.float32),
            jax.ShapeDtypeStruct((NTB * RB, 1), jnp.float32),
            jax.ShapeDtypeStruct(((NPB + 1) * RB, 1), jnp.int32),
            jax.ShapeDtypeStruct((1, 32), jnp.int32),
        ],
        scratch_shapes=[
            pltpu.VMEM((NPB, RB, E), jnp.float32),
            pltpu.VMEM((1, E), jnp.float32),
            pltpu.VMEM((1, E), jnp.float32),
            pltpu.VMEM((1, E), jnp.float32),
        ],
        compiler_params=pltpu.CompilerParams(
            dimension_semantics=("arbitrary", "arbitrary"),
        ),
    )(x, router_w, router_b.reshape(1, E))
    return pa, pb, pos, blk


def _make_dispatch(T, H, CAP, NPW):
    mesh = plsc.VectorSubcoreMesh(core_axis_name="c", subcore_axis_name="s")

    @functools.partial(
        pl.kernel, mesh=mesh,
        out_type=[
            jax.ShapeDtypeStruct((CAP, H), jnp.float32),
            jax.ShapeDtypeStruct((CAP,), jnp.float32),
        ],
        scratch_types=[
            pltpu.VMEM((NPW,), jnp.int32),
            pltpu.VMEM((NPW,), jnp.float32),
            pltpu.VMEM((16,), jnp.int32),
            pltpu.VMEM((16,), jnp.int32),
            pltpu.VMEM((16,), jnp.float32),
            pltpu.VMEM((16, H), jnp.float32),
            pltpu.SemaphoreType.DMA,
        ],
    )
    def dispatch(pos_hbm, p_hbm, x_hbm, xs_hbm, ps_hbm,
                 pos_v, pp_v, tok_v, sc_v, pc_v, xrow_v, sem):
        c = lax.axis_index("c")
        s = lax.axis_index("s")
        wid = c * 16 + s
        pbase = wid * NPW
        lanes = lax.iota(jnp.int32, 16)
        pltpu.sync_copy(pos_hbm.at[pl.ds(pbase, NPW)], pos_v)
        pltpu.sync_copy(p_hbm.at[pl.ds(pbase, NPW)], pp_v)
        for ch in range(NPW // 16):
            sc_v[...] = pos_v[pl.ds(ch * 16, 16)]
            tok_v[...] = (pbase + ch * 16 + lanes) & (T - 1)
            pc_v[...] = pp_v[pl.ds(ch * 16, 16)]
            pltpu.async_copy(x_hbm.at[tok_v], xrow_v, sem).wait()
            pltpu.sync_copy(xrow_v, xs_hbm.at[sc_v])
            pltpu.sync_copy(pc_v, ps_hbm.at[sc_v])

    return dispatch


def _gmm_kernel(be_ref, xs_ref, ps_ref, gup_ref, gbe_ref, gbo_ref, dw_ref,
                db_ref, y_ref, sg_ref, su_ref, wg_ref, wu_ref, last_ref):
    b = pl.program_id(0)
    be = be_ref[b]

    @pl.when(b == 0)
    def _init():
        last_ref[0] = -1
        F2, I = sg_ref.shape
        r = jax.lax.broadcasted_iota(jnp.int32, (F2, I), 0)
        col = jax.lax.broadcasted_iota(jnp.int32, (F2, I), 1)
        sg_ref[...] = (r == 2 * col).astype(jnp.bfloat16)
        su_ref[...] = (r == 2 * col + 1).astype(jnp.bfloat16)

    @pl.when(be >= 0)
    def _compute():
        @pl.when(be != last_ref[0])
        def _split():
            wgu = gup_ref[0].astype(jnp.bfloat16)
            wg_ref[...] = _dot(wgu, sg_ref[...]).astype(jnp.bfloat16)
            wu_ref[...] = _dot(wgu, su_ref[...]).astype(jnp.bfloat16)
            last_ref[0] = be

        xb = xs_ref[...].astype(jnp.bfloat16)
        g = _dot(xb, wg_ref[...]) + gbe_ref[0]
        u = _dot(xb, wu_ref[...]) + gbo_ref[0]
        g = jnp.minimum(g, LIMIT)
        glu = g * jax.nn.sigmoid(g * ALPHA)
        act = (jnp.clip(u, -LIMIT, LIMIT) + 1.0) * glu
        n = xb.shape[0]
        prow = jnp.broadcast_to(ps_ref[0], (n, n))
        diag = jnp.where(
            jax.lax.broadcasted_iota(jnp.int32, (n, n), 0)
            == jax.lax.broadcasted_iota(jnp.int32, (n, n), 1), prow, 0.0)
        prob = _dot(diag, jnp.ones((n, 1), jnp.float32),
                    prec=jax.lax.Precision.HIGHEST)
        y_ref[...] = (_dot(act.astype(jnp.bfloat16),
                           dw_ref[0].astype(jnp.bfloat16)) + db_ref[0]) * prob


def _gmm(xs, ps, blk_e, gate_up_proj, gbe, gbo, down_proj, db3):
    CAP, H = xs.shape
    E, _, F2 = gate_up_proj.shape
    I = F2 // 2
    NB = CAP // BM
    grid_spec = pltpu.PrefetchScalarGridSpec(
        num_scalar_prefetch=1,
        grid=(NB,),
        in_specs=[
            pl.BlockSpec((BM, H), lambda b, be: (jnp.where(be[b] < 0, 0, b), 0)),
            pl.BlockSpec((1, 1, BM), lambda b, be: (jnp.where(be[b] < 0, 0, b), 0, 0)),
            pl.BlockSpec((1, H, F2), lambda b, be: (jnp.maximum(be[b], 0), 0, 0)),
            pl.BlockSpec((1, 1, I), lambda b, be: (jnp.maximum(be[b], 0), 0, 0)),
            pl.BlockSpec((1, 1, I), lambda b, be: (jnp.maximum(be[b], 0), 0, 0)),
            pl.BlockSpec((1, I, H), lambda b, be: (jnp.maximum(be[b], 0), 0, 0)),
            pl.BlockSpec((1, 1, H), lambda b, be: (jnp.maximum(be[b], 0), 0, 0)),
        ],
        out_specs=pl.BlockSpec(
            (BM, H), lambda b, be: (jnp.where(be[b] < 0, NB, b), 0)),
        scratch_shapes=[
            pltpu.VMEM((F2, I), jnp.bfloat16),
            pltpu.VMEM((F2, I), jnp.bfloat16),
            pltpu.VMEM((H, I), jnp.bfloat16),
            pltpu.VMEM((H, I), jnp.bfloat16),
            pltpu.SMEM((1,), jnp.int32),
        ],
    )
    return pl.pallas_call(
        _gmm_kernel,
        grid_spec=grid_spec,
        out_shape=jax.ShapeDtypeStruct(((NB + 1) * BM, H), jnp.float32),
        compiler_params=pltpu.CompilerParams(
            dimension_semantics=("arbitrary",),
            vmem_limit_bytes=100 * 1024 * 1024,
        ),
    )(blk_e, xs, ps, gate_up_proj, gbe, gbo, down_proj, db3)


def _make_combine(T, H, NTW):
    mesh = plsc.VectorSubcoreMesh(core_axis_name="c", subcore_axis_name="s")

    @functools.partial(
        pl.kernel, mesh=mesh,
        out_type=[
            jax.ShapeDtypeStruct((T, H), jnp.float32),
            jax.ShapeDtypeStruct((T, H), jnp.float32),
        ],
        scratch_types=[
            pltpu.VMEM((NTW,), jnp.int32),
            pltpu.VMEM((NTW,), jnp.int32),
            pltpu.VMEM((16,), jnp.int32),
            pltpu.VMEM((16, H), jnp.float32),
            pltpu.SemaphoreType.DMA,
        ],
    )
    def combine(pos_hbm, y_hbm, oa_hbm, ob_hbm,
                pa_v, pb_v, gi_v, ya_v, sem):
        c = lax.axis_index("c")
        s = lax.axis_index("s")
        wid = c * 16 + s
        tbase = wid * NTW
        pltpu.sync_copy(pos_hbm.at[pl.ds(tbase, NTW)], pa_v)
        pltpu.sync_copy(pos_hbm.at[pl.ds(T + tbase, NTW)], pb_v)
        for ch in range(NTW // 16):
            gi_v[...] = pa_v[pl.ds(ch * 16, 16)]
            pltpu.async_copy(y_hbm.at[gi_v], ya_v, sem).wait()
            pltpu.sync_copy(ya_v, oa_hbm.at[pl.ds(tbase + ch * 16, 16)])
            gi_v[...] = pb_v[pl.ds(ch * 16, 16)]
            pltpu.async_copy(y_hbm.at[gi_v], ya_v, sem).wait()
            pltpu.sync_copy(ya_v, ob_hbm.at[pl.ds(tbase + ch * 16, 16)])

    return combine


def _add_kernel(a_ref, b_ref, o_ref):
    o_ref[...] = a_ref[...] + b_ref[...]


def _tc_add(a, b):
    T, H = a.shape
    return pl.pallas_call(
        _add_kernel,
        grid=(T // BM,),
        in_specs=[
            pl.BlockSpec((BM, H), lambda m: (m, 0)),
            pl.BlockSpec((BM, H), lambda m: (m, 0)),
        ],
        out_specs=pl.BlockSpec((BM, H), lambda m: (m, 0)),
        out_shape=jax.ShapeDtypeStruct((T, H), jnp.float32),
    )(a, b)


def kernel(hidden_states, router_w, router_b, gate_up_proj, gate_up_bias,
           down_proj, down_bias):
    B, S, H = hidden_states.shape
    E, _, F2 = gate_up_proj.shape
    I = F2 // 2
    T = B * S
    x = hidden_states.reshape(T, H)
    CAP = 2 * T + E * BM
    NB = CAP // BM

    pa2, pb2, pos2, blk2 = _router_sort(x, router_w, router_b, NB)
    p_flat = jnp.concatenate([pa2.reshape(-1), pb2.reshape(-1)])
    pos_flat = pos2.reshape(-1)
    blk_e = blk2.reshape(32)

    dispatch = _make_dispatch(T, H, CAP, 2 * T // NSC)
    xs, ps = dispatch(pos_flat, p_flat, x)
    ps = ps.reshape(CAP // BM, 1, BM)

    gbe = gate_up_bias[:, 0::2].reshape(E, 1, I)
    gbo = gate_up_bias[:, 1::2].reshape(E, 1, I)
    db3 = down_bias.reshape(E, 1, H)
    y = _gmm(xs, ps, blk_e, gate_up_proj, gbe, gbo, down_proj, db3)

    combine = _make_combine(T, H, T // NSC)
    oa, ob = combine(pos_flat, y)
    out = _tc_add(oa, ob)
    return out.reshape(B, S, H)


# speedup vs baseline: 1.0695x; 1.0695x over previous
import functools

import jax
import jax.numpy as jnp
from jax import lax
from jax.experimental import pallas as pl
from jax.experimental.pallas import tpu as pltpu
from jax.experimental.pallas import tpu_sc as plsc

ALPHA = 1.702
LIMIT = 7.0
BM = 256
RB = 256
NSC = 32


def _dot(a, b, prec=None):
    return jax.lax.dot_general(
        a, b, (((1,), (0,)), ((), ())),
        precision=prec, preferred_element_type=jnp.float32)


def _rs_kernel(x_ref, rw_ref, rb_ref, pa_ref, pb_ref, pos_ref, blk_ref,
               os_ref, carry_ref, carry2_ref, start_ref):
    ph = pl.program_id(0)
    m = pl.program_id(1)
    E = rw_ref.shape[1]
    NTB = os_ref.shape[0] // 2

    @pl.when((ph == 0) & (m < NTB))
    def _router():
        l = _dot(x_ref[...], rw_ref[...]) + rb_ref[...]
        m1 = jnp.max(l, axis=1, keepdims=True)
        tri = (jax.lax.broadcasted_iota(jnp.int32, (E, E), 0)
               <= jax.lax.broadcasted_iota(jnp.int32, (E, E), 1)).astype(jnp.float32)
        is1 = (l == m1).astype(jnp.float32)
        sel1 = is1 * (_dot(is1, tri) == 1.0).astype(jnp.float32)
        l2 = jnp.where(sel1 > 0.0, -jnp.inf, l)
        m2 = jnp.max(l2, axis=1, keepdims=True)
        is2 = (l2 == m2).astype(jnp.float32)
        sel2 = is2 * (_dot(is2, tri) == 1.0).astype(jnp.float32)
        e2 = jnp.exp(m2 - m1)
        p1 = 1.0 / (1.0 + e2)
        p2 = e2 / (1.0 + e2)
        os_ref[pl.ds(m, 1)] = sel1.reshape(1, RB, E)
        os_ref[pl.ds(m + NTB, 1)] = sel2.reshape(1, RB, E)
        pa_ref[...] = p1
        pb_ref[...] = p2

    @pl.when(ph == 0)
    def _count():
        @pl.when(m == 0)
        def _zero():
            carry_ref[...] = jnp.zeros_like(carry_ref)

        ob = os_ref[pl.ds(m, 1)].reshape(RB, E)
        carry_ref[...] = carry_ref[...] + jnp.sum(ob, axis=0, keepdims=True)

    @pl.when(ph == 1)
    def _pos():
        E_ = carry_ref.shape[1]

        @pl.when(m == 0)
        def _starts():
            carry2_ref[...] = jnp.zeros_like(carry2_ref)
            tot = carry_ref[...]
            ptot = jnp.floor((tot + (BM - 1.0)) * (1.0 / BM)) * BM
            u8s = (jax.lax.broadcasted_iota(jnp.int32, (E_, E_), 0)
                   < jax.lax.broadcasted_iota(jnp.int32, (E_, E_), 1)).astype(jnp.float32)
            start = _dot(ptot, u8s)
            start_ref[...] = start
            nlan = blk_ref.shape[1]
            rows0 = jax.lax.broadcasted_iota(
                jnp.int32, (1, nlan), 1).astype(jnp.float32) * BM
            blk = jnp.full((1, nlan), -1, jnp.int32)
            for e in range(E_):
                rep = (jax.lax.broadcasted_iota(jnp.int32, (E_, nlan), 0)
                       == e).astype(jnp.float32)
                se = _dot(start, rep)
                pe = _dot(ptot, rep)
                mm = (rows0 >= se) & (rows0 < se + pe)
                blk = jnp.where(mm, e, blk)
            blk_ref[...] = blk

        ob = os_ref[pl.ds(m, 1)].reshape(RB, E)
        ltri = (jax.lax.broadcasted_iota(jnp.int32, (RB, RB), 0)
                > jax.lax.broadcasted_iota(jnp.int32, (RB, RB), 1)).astype(jnp.float32)
        pre = _dot(ltri, ob) + carry2_ref[...]
        rank = jnp.sum(ob * pre, axis=1, keepdims=True)
        carry2_ref[...] = carry2_ref[...] + jnp.sum(ob, axis=0, keepdims=True)
        sv = jax.lax.dot_general(ob, start_ref[...], (((1,), (1,)), ((), ())),
                                 preferred_element_type=jnp.float32)
        pos_ref[...] = (sv + rank).astype(jnp.int32)


def _router_sort(x, router_w, router_b, NB):
    T, H = x.shape
    E = router_w.shape[1]
    NTB = T // RB
    NPB = 2 * NTB
    pa, pb, pos, blk = pl.pallas_call(
        _rs_kernel,
        grid=(2, NPB),
        in_specs=[
            pl.BlockSpec((RB, H), lambda ph, m: (
                jnp.where(ph == 0, jnp.minimum(m, NTB - 1), NTB - 1), 0)),
            pl.BlockSpec((H, E), lambda ph, m: (0, 0)),
            pl.BlockSpec((1, E), lambda ph, m: (0, 0)),
        ],
        out_specs=[
            pl.BlockSpec((RB, 1), lambda ph, m: (
                jnp.where(ph == 0, jnp.minimum(m, NTB - 1), NTB - 1), 0)),
            pl.BlockSpec((RB, 1), lambda ph, m: (
                jnp.where(ph == 0, jnp.minimum(m, NTB - 1), NTB - 1), 0)),
            pl.BlockSpec((RB, 1), lambda ph, m: (
                jnp.where(ph == 1, m, NPB), 0)),
            pl.BlockSpec((1, 32), lambda ph, m: (0, 0)),
        ],
        out_shape=[
            jax.ShapeDtypeStruct((NTB * RB, 1), jnp.float32),
            jax.ShapeDtypeStruct((NTB * RB, 1), jnp.float32),
            jax.ShapeDtypeStruct(((NPB + 1) * RB, 1), jnp.int32),
            jax.ShapeDtypeStruct((1, 32), jnp.int32),
        ],
        scratch_shapes=[
            pltpu.VMEM((NPB, RB, E), jnp.float32),
            pltpu.VMEM((1, E), jnp.float32),
            pltpu.VMEM((1, E), jnp.float32),
            pltpu.VMEM((1, E), jnp.float32),
        ],
        compiler_params=pltpu.CompilerParams(
            dimension_semantics=("arbitrary", "arbitrary"),
        ),
    )(x, router_w, router_b.reshape(1, E))
    return pa, pb, pos, blk


def _make_dispatch(T, H, CAP, NPW):
    mesh = plsc.VectorSubcoreMesh(core_axis_name="c", subcore_axis_name="s")

    @functools.partial(
        pl.kernel, mesh=mesh,
        out_type=[
            jax.ShapeDtypeStruct((CAP, H), jnp.float32),
            jax.ShapeDtypeStruct((CAP,), jnp.float32),
        ],
        scratch_types=[
            pltpu.VMEM((NPW,), jnp.int32),
            pltpu.VMEM((NPW,), jnp.float32),
            pltpu.VMEM((16,), jnp.int32),
            pltpu.VMEM((16,), jnp.int32),
            pltpu.VMEM((16,), jnp.int32),
            pltpu.VMEM((16,), jnp.float32),
            pltpu.VMEM((16, H), jnp.float32),
            pltpu.VMEM((16, H), jnp.float32),
            pltpu.SemaphoreType.DMA,
            pltpu.SemaphoreType.DMA,
        ],
    )
    def dispatch(pos_hbm, p_hbm, x_hbm, xs_hbm, ps_hbm,
                 pos_v, pp_v, tok0_v, tok1_v, sc_v, pc_v, xr0_v, xr1_v,
                 sem0, sem1):
        c = lax.axis_index("c")
        s = lax.axis_index("s")
        wid = c * 16 + s
        pbase = wid * NPW
        lanes = lax.iota(jnp.int32, 16)
        pltpu.sync_copy(pos_hbm.at[pl.ds(pbase, NPW)], pos_v)
        pltpu.sync_copy(p_hbm.at[pl.ds(pbase, NPW)], pp_v)
        nch = NPW // 16
        toks = [tok0_v, tok1_v]
        bufs = [xr0_v, xr1_v]
        sems = [sem0, sem1]
        tok0_v[...] = (pbase + lanes) & (T - 1)
        cp0 = pltpu.async_copy(x_hbm.at[tok0_v], xr0_v, sem0)
        copies = [cp0, None]
        for ch in range(nch):
            cur = ch & 1
            nxt = cur ^ 1
            if ch + 1 < nch:
                toks[nxt][...] = (pbase + (ch + 1) * 16 + lanes) & (T - 1)
                copies[nxt] = pltpu.async_copy(
                    x_hbm.at[toks[nxt]], bufs[nxt], sems[nxt])
            copies[cur].wait()
            sc_v[...] = pos_v[pl.ds(ch * 16, 16)]
            pc_v[...] = pp_v[pl.ds(ch * 16, 16)]
            pltpu.sync_copy(bufs[cur], xs_hbm.at[sc_v])
            pltpu.sync_copy(pc_v, ps_hbm.at[sc_v])

    return dispatch


def _gmm_kernel(be_ref, xs_ref, ps_ref, gup_ref, gbe_ref, gbo_ref, dw_ref,
                db_ref, y_ref, sg_ref, su_ref, wg_ref, wu_ref, last_ref):
    b = pl.program_id(0)
    be = be_ref[b]

    @pl.when(b == 0)
    def _init():
        last_ref[0] = -1
        F2, I = sg_ref.shape
        r = jax.lax.broadcasted_iota(jnp.int32, (F2, I), 0)
        col = jax.lax.broadcasted_iota(jnp.int32, (F2, I), 1)
        sg_ref[...] = (r == 2 * col).astype(jnp.bfloat16)
        su_ref[...] = (r == 2 * col + 1).astype(jnp.bfloat16)

    @pl.when(be >= 0)
    def _compute():
        @pl.when(be != last_ref[0])
        def _split():
            wgu = gup_ref[0].astype(jnp.bfloat16)
            wg_ref[...] = _dot(wgu, sg_ref[...]).astype(jnp.bfloat16)
            wu_ref[...] = _dot(wgu, su_ref[...]).astype(jnp.bfloat16)
            last_ref[0] = be

        xb = xs_ref[...].astype(jnp.bfloat16)
        g = _dot(xb, wg_ref[...]) + gbe_ref[0]
        u = _dot(xb, wu_ref[...]) + gbo_ref[0]
        g = jnp.minimum(g, LIMIT)
        glu = g * jax.nn.sigmoid(g * ALPHA)
        act = (jnp.clip(u, -LIMIT, LIMIT) + 1.0) * glu
        n = xb.shape[0]
        prow = jnp.broadcast_to(ps_ref[0], (n, n))
        diag = jnp.where(
            jax.lax.broadcasted_iota(jnp.int32, (n, n), 0)
            == jax.lax.broadcasted_iota(jnp.int32, (n, n), 1), prow, 0.0)
        prob = _dot(diag, jnp.ones((n, 1), jnp.float32),
                    prec=jax.lax.Precision.HIGHEST)
        y_ref[...] = (_dot(act.astype(jnp.bfloat16),
                           dw_ref[0].astype(jnp.bfloat16)) + db_ref[0]) * prob


def _gmm(xs, ps, blk_e, gate_up_proj, gbe, gbo, down_proj, db3):
    CAP, H = xs.shape
    E, _, F2 = gate_up_proj.shape
    I = F2 // 2
    NB = CAP // BM
    grid_spec = pltpu.PrefetchScalarGridSpec(
        num_scalar_prefetch=1,
        grid=(NB,),
        in_specs=[
            pl.BlockSpec((BM, H), lambda b, be: (jnp.where(be[b] < 0, 0, b), 0)),
            pl.BlockSpec((1, 1, BM), lambda b, be: (jnp.where(be[b] < 0, 0, b), 0, 0)),
            pl.BlockSpec((1, H, F2), lambda b, be: (jnp.maximum(be[b], 0), 0, 0)),
            pl.BlockSpec((1, 1, I), lambda b, be: (jnp.maximum(be[b], 0), 0, 0)),
            pl.BlockSpec((1, 1, I), lambda b, be: (jnp.maximum(be[b], 0), 0, 0)),
            pl.BlockSpec((1, I, H), lambda b, be: (jnp.maximum(be[b], 0), 0, 0)),
            pl.BlockSpec((1, 1, H), lambda b, be: (jnp.maximum(be[b], 0), 0, 0)),
        ],
        out_specs=pl.BlockSpec(
            (BM, H), lambda b, be: (jnp.where(be[b] < 0, NB, b), 0)),
        scratch_shapes=[
            pltpu.VMEM((F2, I), jnp.bfloat16),
            pltpu.VMEM((F2, I), jnp.bfloat16),
            pltpu.VMEM((H, I), jnp.bfloat16),
            pltpu.VMEM((H, I), jnp.bfloat16),
            pltpu.SMEM((1,), jnp.int32),
        ],
    )
    return pl.pallas_call(
        _gmm_kernel,
        grid_spec=grid_spec,
        out_shape=jax.ShapeDtypeStruct(((NB + 1) * BM, H), jnp.float32),
        compiler_params=pltpu.CompilerParams(
            dimension_semantics=("arbitrary",),
            vmem_limit_bytes=100 * 1024 * 1024,
        ),
    )(blk_e, xs, ps, gate_up_proj, gbe, gbo, down_proj, db3)


def _make_combine(T, H, NTW):
    mesh = plsc.VectorSubcoreMesh(core_axis_name="c", subcore_axis_name="s")

    @functools.partial(
        pl.kernel, mesh=mesh,
        out_type=[
            jax.ShapeDtypeStruct((T, H), jnp.float32),
            jax.ShapeDtypeStruct((T, H), jnp.float32),
        ],
        scratch_types=[
            pltpu.VMEM((NTW,), jnp.int32),
            pltpu.VMEM((NTW,), jnp.int32),
            pltpu.VMEM((16,), jnp.int32),
            pltpu.VMEM((16, H), jnp.float32),
            pltpu.SemaphoreType.DMA,
        ],
    )
    def combine(pos_hbm, y_hbm, oa_hbm, ob_hbm,
                pa_v, pb_v, gi_v, ya_v, sem):
        c = lax.axis_index("c")
        s = lax.axis_index("s")
        wid = c * 16 + s
        tbase = wid * NTW
        pltpu.sync_copy(pos_hbm.at[pl.ds(tbase, NTW)], pa_v)
        pltpu.sync_copy(pos_hbm.at[pl.ds(T + tbase, NTW)], pb_v)
        for ch in range(NTW // 16):
            gi_v[...] = pa_v[pl.ds(ch * 16, 16)]
            pltpu.async_copy(y_hbm.at[gi_v], ya_v, sem).wait()
            pltpu.sync_copy(ya_v, oa_hbm.at[pl.ds(tbase + ch * 16, 16)])
            gi_v[...] = pb_v[pl.ds(ch * 16, 16)]
            pltpu.async_copy(y_hbm.at[gi_v], ya_v, sem).wait()
            pltpu.sync_copy(ya_v, ob_hbm.at[pl.ds(tbase + ch * 16, 16)])

    return combine


def _add_kernel(a_ref, b_ref, o_ref):
    o_ref[...] = a_ref[...] + b_ref[...]


def _tc_add(a, b):
    T, H = a.shape
    return pl.pallas_call(
        _add_kernel,
        grid=(T // BM,),
        in_specs=[
            pl.BlockSpec((BM, H), lambda m: (m, 0)),
            pl.BlockSpec((BM, H), lambda m: (m, 0)),
        ],
        out_specs=pl.BlockSpec((BM, H), lambda m: (m, 0)),
        out_shape=jax.ShapeDtypeStruct((T, H), jnp.float32),
    )(a, b)


def kernel(hidden_states, router_w, router_b, gate_up_proj, gate_up_bias,
           down_proj, down_bias):
    B, S, H = hidden_states.shape
    E, _, F2 = gate_up_proj.shape
    I = F2 // 2
    T = B * S
    x = hidden_states.reshape(T, H)
    CAP = 2 * T + E * BM
    NB = CAP // BM

    pa2, pb2, pos2, blk2 = _router_sort(x, router_w, router_b, NB)
    p_flat = jnp.concatenate([pa2.reshape(-1), pb2.reshape(-1)])
    pos_flat = pos2.reshape(-1)
    blk_e = blk2.reshape(32)

    dispatch = _make_dispatch(T, H, CAP, 2 * T // NSC)
    xs, ps = dispatch(pos_flat, p_flat, x)
    ps = ps.reshape(CAP // BM, 1, BM)

    gbe = gate_up_bias[:, 0::2].reshape(E, 1, I)
    gbo = gate_up_bias[:, 1::2].reshape(E, 1, I)
    db3 = down_bias.reshape(E, 1, H)
    y = _gmm(xs, ps, blk_e, gate_up_proj, gbe, gbo, down_proj, db3)

    combine = _make_combine(T, H, T // NSC)
    oa, ob = combine(pos_flat, y)
    out = _tc_add(oa, ob)
    return out.reshape(B, S, H)


# speedup vs baseline: 1.0927x; 1.0217x over previous
import functools

import jax
import jax.numpy as jnp
from jax import lax
from jax.experimental import pallas as pl
from jax.experimental.pallas import tpu as pltpu
from jax.experimental.pallas import tpu_sc as plsc

ALPHA = 1.702
LIMIT = 7.0
BM = 256
RB = 256
NSC = 32


def _dot(a, b, prec=None):
    return jax.lax.dot_general(
        a, b, (((1,), (0,)), ((), ())),
        precision=prec, preferred_element_type=jnp.float32)


def _rs_kernel(x_ref, rw_ref, rb_ref, pa_ref, pb_ref, pos_ref, blk_ref,
               os_ref, carry_ref, carry2_ref, start_ref):
    ph = pl.program_id(0)
    m = pl.program_id(1)
    E = rw_ref.shape[1]
    NTB = os_ref.shape[0] // 2

    @pl.when((ph == 0) & (m < NTB))
    def _router():
        l = _dot(x_ref[...], rw_ref[...]) + rb_ref[...]
        m1 = jnp.max(l, axis=1, keepdims=True)
        tri = (jax.lax.broadcasted_iota(jnp.int32, (E, E), 0)
               <= jax.lax.broadcasted_iota(jnp.int32, (E, E), 1)).astype(jnp.float32)
        is1 = (l == m1).astype(jnp.float32)
        sel1 = is1 * (_dot(is1, tri) == 1.0).astype(jnp.float32)
        l2 = jnp.where(sel1 > 0.0, -jnp.inf, l)
        m2 = jnp.max(l2, axis=1, keepdims=True)
        is2 = (l2 == m2).astype(jnp.float32)
        sel2 = is2 * (_dot(is2, tri) == 1.0).astype(jnp.float32)
        e2 = jnp.exp(m2 - m1)
        p1 = 1.0 / (1.0 + e2)
        p2 = e2 / (1.0 + e2)
        os_ref[pl.ds(m, 1)] = sel1.reshape(1, RB, E)
        os_ref[pl.ds(m + NTB, 1)] = sel2.reshape(1, RB, E)
        pa_ref[...] = p1
        pb_ref[...] = p2

    @pl.when(ph == 0)
    def _count():
        @pl.when(m == 0)
        def _zero():
            carry_ref[...] = jnp.zeros_like(carry_ref)

        ob = os_ref[pl.ds(m, 1)].reshape(RB, E)
        carry_ref[...] = carry_ref[...] + jnp.sum(ob, axis=0, keepdims=True)

    @pl.when(ph == 1)
    def _pos():
        E_ = carry_ref.shape[1]

        @pl.when(m == 0)
        def _starts():
            carry2_ref[...] = jnp.zeros_like(carry2_ref)
            tot = carry_ref[...]
            ptot = jnp.floor((tot + (BM - 1.0)) * (1.0 / BM)) * BM
            u8s = (jax.lax.broadcasted_iota(jnp.int32, (E_, E_), 0)
                   < jax.lax.broadcasted_iota(jnp.int32, (E_, E_), 1)).astype(jnp.float32)
            start = _dot(ptot, u8s)
            start_ref[...] = start
            nlan = blk_ref.shape[1]
            rows0 = jax.lax.broadcasted_iota(
                jnp.int32, (1, nlan), 1).astype(jnp.float32) * BM
            blk = jnp.full((1, nlan), -1, jnp.int32)
            for e in range(E_):
                rep = (jax.lax.broadcasted_iota(jnp.int32, (E_, nlan), 0)
                       == e).astype(jnp.float32)
                se = _dot(start, rep)
                pe = _dot(ptot, rep)
                mm = (rows0 >= se) & (rows0 < se + pe)
                blk = jnp.where(mm, e, blk)
            blk_ref[...] = blk

        ob = os_ref[pl.ds(m, 1)].reshape(RB, E)
        ltri = (jax.lax.broadcasted_iota(jnp.int32, (RB, RB), 0)
                > jax.lax.broadcasted_iota(jnp.int32, (RB, RB), 1)).astype(jnp.float32)
        pre = _dot(ltri, ob) + carry2_ref[...]
        rank = jnp.sum(ob * pre, axis=1, keepdims=True)
        carry2_ref[...] = carry2_ref[...] + jnp.sum(ob, axis=0, keepdims=True)
        sv = jax.lax.dot_general(ob, start_ref[...], (((1,), (1,)), ((), ())),
                                 preferred_element_type=jnp.float32)
        pos_ref[...] = (sv + rank).astype(jnp.int32)


def _router_sort(x, router_w, router_b, NB):
    T, H = x.shape
    E = router_w.shape[1]
    NTB = T // RB
    NPB = 2 * NTB
    pa, pb, pos, blk = pl.pallas_call(
        _rs_kernel,
        grid=(2, NPB),
        in_specs=[
            pl.BlockSpec((RB, H), lambda ph, m: (
                jnp.where(ph == 0, jnp.minimum(m, NTB - 1), NTB - 1), 0)),
            pl.BlockSpec((H, E), lambda ph, m: (0, 0)),
            pl.BlockSpec((1, E), lambda ph, m: (0, 0)),
        ],
        out_specs=[
            pl.BlockSpec((RB, 1), lambda ph, m: (
                jnp.where(ph == 0, jnp.minimum(m, NTB - 1), NTB - 1), 0)),
            pl.BlockSpec((RB, 1), lambda ph, m: (
                jnp.where(ph == 0, jnp.minimum(m, NTB - 1), NTB - 1), 0)),
            pl.BlockSpec((RB, 1), lambda ph, m: (
                jnp.where(ph == 1, m, NPB), 0)),
            pl.BlockSpec((1, 32), lambda ph, m: (0, 0)),
        ],
        out_shape=[
            jax.ShapeDtypeStruct((NTB * RB, 1), jnp.float32),
            jax.ShapeDtypeStruct((NTB * RB, 1), jnp.float32),
            jax.ShapeDtypeStruct(((NPB + 1) * RB, 1), jnp.int32),
            jax.ShapeDtypeStruct((1, 32), jnp.int32),
        ],
        scratch_shapes=[
            pltpu.VMEM((NPB, RB, E), jnp.float32),
            pltpu.VMEM((1, E), jnp.float32),
            pltpu.VMEM((1, E), jnp.float32),
            pltpu.VMEM((1, E), jnp.float32),
        ],
        compiler_params=pltpu.CompilerParams(
            dimension_semantics=("arbitrary", "arbitrary"),
        ),
    )(x, router_w, router_b.reshape(1, E))
    return pa, pb, pos, blk


def _make_dispatch(T, H, CAP, NPW):
    mesh = plsc.VectorSubcoreMesh(core_axis_name="c", subcore_axis_name="s")

    @functools.partial(
        pl.kernel, mesh=mesh,
        out_type=[
            jax.ShapeDtypeStruct((CAP, H), jnp.float32),
            jax.ShapeDtypeStruct((CAP,), jnp.float32),
        ],
        scratch_types=[
            pltpu.VMEM((NPW,), jnp.int32),
            pltpu.VMEM((NPW,), jnp.float32),
            pltpu.VMEM((16,), jnp.int32),
            pltpu.VMEM((16,), jnp.int32),
            pltpu.VMEM((16,), jnp.int32),
            pltpu.VMEM((16,), jnp.float32),
            pltpu.VMEM((16, H), jnp.float32),
            pltpu.VMEM((16, H), jnp.float32),
            pltpu.SemaphoreType.DMA,
            pltpu.SemaphoreType.DMA,
        ],
    )
    def dispatch(pos_hbm, p_hbm, x_hbm, xs_hbm, ps_hbm,
                 pos_v, pp_v, tok0_v, tok1_v, sc_v, pc_v, xr0_v, xr1_v,
                 sem0, sem1):
        c = lax.axis_index("c")
        s = lax.axis_index("s")
        wid = c * 16 + s
        pbase = wid * NPW
        lanes = lax.iota(jnp.int32, 16)
        pltpu.sync_copy(pos_hbm.at[pl.ds(pbase, NPW)], pos_v)
        pltpu.sync_copy(p_hbm.at[pl.ds(pbase, NPW)], pp_v)
        nch = NPW // 16
        toks = [tok0_v, tok1_v]
        bufs = [xr0_v, xr1_v]
        sems = [sem0, sem1]
        tok0_v[...] = (pbase + lanes) & (T - 1)
        cp0 = pltpu.async_copy(x_hbm.at[tok0_v], xr0_v, sem0)
        copies = [cp0, None]
        for ch in range(nch):
            cur = ch & 1
            nxt = cur ^ 1
            if ch + 1 < nch:
                toks[nxt][...] = (pbase + (ch + 1) * 16 + lanes) & (T - 1)
                copies[nxt] = pltpu.async_copy(
                    x_hbm.at[toks[nxt]], bufs[nxt], sems[nxt])
            copies[cur].wait()
            sc_v[...] = pos_v[pl.ds(ch * 16, 16)]
            pc_v[...] = pp_v[pl.ds(ch * 16, 16)]
            pltpu.sync_copy(bufs[cur], xs_hbm.at[sc_v])
            pltpu.sync_copy(pc_v, ps_hbm.at[sc_v])

    return dispatch


def _gmm_kernel(be_ref, xs_ref, ps_ref, gup_ref, gbe_ref, gbo_ref, dw_ref,
                db_ref, y_ref, sg_ref, su_ref, wg_ref, wu_ref, last_ref):
    b = pl.program_id(0)
    be = be_ref[b]

    @pl.when(b == 0)
    def _init():
        last_ref[0] = -1
        F2, I = sg_ref.shape
        r = jax.lax.broadcasted_iota(jnp.int32, (F2, I), 0)
        col = jax.lax.broadcasted_iota(jnp.int32, (F2, I), 1)
        sg_ref[...] = (r == 2 * col).astype(jnp.bfloat16)
        su_ref[...] = (r == 2 * col + 1).astype(jnp.bfloat16)

    @pl.when(be >= 0)
    def _compute():
        @pl.when(be != last_ref[0])
        def _split():
            wgu = gup_ref[0].astype(jnp.bfloat16)
            wg_ref[...] = _dot(wgu, sg_ref[...]).astype(jnp.bfloat16)
            wu_ref[...] = _dot(wgu, su_ref[...]).astype(jnp.bfloat16)
            last_ref[0] = be

        xb = xs_ref[...].astype(jnp.bfloat16)
        g = _dot(xb, wg_ref[...]) + gbe_ref[0]
        u = _dot(xb, wu_ref[...]) + gbo_ref[0]
        g = jnp.minimum(g, LIMIT)
        glu = g * jax.nn.sigmoid(g * ALPHA)
        act = (jnp.clip(u, -LIMIT, LIMIT) + 1.0) * glu
        n = xb.shape[0]
        prow = jnp.broadcast_to(ps_ref[0], (n, n))
        diag = jnp.where(
            jax.lax.broadcasted_iota(jnp.int32, (n, n), 0)
            == jax.lax.broadcasted_iota(jnp.int32, (n, n), 1), prow, 0.0)
        prob = _dot(diag, jnp.ones((n, 1), jnp.float32),
                    prec=jax.lax.Precision.HIGHEST)
        y_ref[...] = (_dot(act.astype(jnp.bfloat16),
                           dw_ref[0].astype(jnp.bfloat16)) + db_ref[0]) * prob


def _gmm(xs, ps, blk_e, gate_up_proj, gbe, gbo, down_proj, db3):
    CAP, H = xs.shape
    E, _, F2 = gate_up_proj.shape
    I = F2 // 2
    NB = CAP // BM
    grid_spec = pltpu.PrefetchScalarGridSpec(
        num_scalar_prefetch=1,
        grid=(NB,),
        in_specs=[
            pl.BlockSpec((BM, H), lambda b, be: (jnp.where(be[b] < 0, 0, b), 0)),
            pl.BlockSpec((1, 1, BM), lambda b, be: (jnp.where(be[b] < 0, 0, b), 0, 0)),
            pl.BlockSpec((1, H, F2), lambda b, be: (jnp.maximum(be[b], 0), 0, 0)),
            pl.BlockSpec((1, 1, I), lambda b, be: (jnp.maximum(be[b], 0), 0, 0)),
            pl.BlockSpec((1, 1, I), lambda b, be: (jnp.maximum(be[b], 0), 0, 0)),
            pl.BlockSpec((1, I, H), lambda b, be: (jnp.maximum(be[b], 0), 0, 0)),
            pl.BlockSpec((1, 1, H), lambda b, be: (jnp.maximum(be[b], 0), 0, 0)),
        ],
        out_specs=pl.BlockSpec(
            (BM, H), lambda b, be: (jnp.where(be[b] < 0, NB, b), 0)),
        scratch_shapes=[
            pltpu.VMEM((F2, I), jnp.bfloat16),
            pltpu.VMEM((F2, I), jnp.bfloat16),
            pltpu.VMEM((H, I), jnp.bfloat16),
            pltpu.VMEM((H, I), jnp.bfloat16),
            pltpu.SMEM((1,), jnp.int32),
        ],
    )
    return pl.pallas_call(
        _gmm_kernel,
        grid_spec=grid_spec,
        out_shape=jax.ShapeDtypeStruct(((NB + 1) * BM, H), jnp.float32),
        compiler_params=pltpu.CompilerParams(
            dimension_semantics=("arbitrary",),
            vmem_limit_bytes=100 * 1024 * 1024,
        ),
    )(blk_e, xs, ps, gate_up_proj, gbe, gbo, down_proj, db3)


def _make_combine(T, H, NTW):
    mesh = plsc.VectorSubcoreMesh(core_axis_name="c", subcore_axis_name="s")

    @functools.partial(
        pl.kernel, mesh=mesh,
        out_type=[
            jax.ShapeDtypeStruct((T, H), jnp.float32),
            jax.ShapeDtypeStruct((T, H), jnp.float32),
        ],
        scratch_types=[
            pltpu.VMEM((NTW,), jnp.int32),
            pltpu.VMEM((NTW,), jnp.int32),
            pltpu.VMEM((16,), jnp.int32),
            pltpu.VMEM((16,), jnp.int32),
            pltpu.VMEM((16, H), jnp.float32),
            pltpu.VMEM((16, H), jnp.float32),
            pltpu.SemaphoreType.DMA,
            pltpu.SemaphoreType.DMA,
        ],
    )
    def combine(pos_hbm, y_hbm, oa_hbm, ob_hbm,
                pa_v, pb_v, gi0_v, gi1_v, yr0_v, yr1_v, sem0, sem1):
        c = lax.axis_index("c")
        s = lax.axis_index("s")
        wid = c * 16 + s
        tbase = wid * NTW
        pltpu.sync_copy(pos_hbm.at[pl.ds(tbase, NTW)], pa_v)
        pltpu.sync_copy(pos_hbm.at[pl.ds(T + tbase, NTW)], pb_v)
        nch = NTW // 16
        gis = [gi0_v, gi1_v]
        bufs = [yr0_v, yr1_v]
        sems = [sem0, sem1]
        outs = []
        for ch in range(nch):
            outs.append((pa_v, ch, oa_hbm))
        for ch in range(nch):
            outs.append((pb_v, ch, ob_hbm))
        src0, ch0, _ = outs[0]
        gi0_v[...] = src0[pl.ds(ch0 * 16, 16)]
        copies = [pltpu.async_copy(y_hbm.at[gi0_v], yr0_v, sem0), None]
        for i, (src, ch, dst) in enumerate(outs):
            cur = i & 1
            nxt = cur ^ 1
            if i + 1 < len(outs):
                nsrc, nch_, _ = outs[i + 1]
                gis[nxt][...] = nsrc[pl.ds(nch_ * 16, 16)]
                copies[nxt] = pltpu.async_copy(
                    y_hbm.at[gis[nxt]], bufs[nxt], sems[nxt])
            copies[cur].wait()
            pltpu.sync_copy(bufs[cur], dst.at[pl.ds(tbase + ch * 16, 16)])

    return combine


def _add_kernel(a_ref, b_ref, o_ref):
    o_ref[...] = a_ref[...] + b_ref[...]


def _tc_add(a, b):
    T, H = a.shape
    return pl.pallas_call(
        _add_kernel,
        grid=(T // BM,),
        in_specs=[
            pl.BlockSpec((BM, H), lambda m: (m, 0)),
            pl.BlockSpec((BM, H), lambda m: (m, 0)),
        ],
        out_specs=pl.BlockSpec((BM, H), lambda m: (m, 0)),
        out_shape=jax.ShapeDtypeStruct((T, H), jnp.float32),
    )(a, b)


def kernel(hidden_states, router_w, router_b, gate_up_proj, gate_up_bias,
           down_proj, down_bias):
    B, S, H = hidden_states.shape
    E, _, F2 = gate_up_proj.shape
    I = F2 // 2
    T = B * S
    x = hidden_states.reshape(T, H)
    CAP = 2 * T + E * BM
    NB = CAP // BM

    pa2, pb2, pos2, blk2 = _router_sort(x, router_w, router_b, NB)
    p_flat = jnp.concatenate([pa2.reshape(-1), pb2.reshape(-1)])
    pos_flat = pos2.reshape(-1)
    blk_e = blk2.reshape(32)

    dispatch = _make_dispatch(T, H, CAP, 2 * T // NSC)
    xs, ps = dispatch(pos_flat, p_flat, x)
    ps = ps.reshape(CAP // BM, 1, BM)

    gbe = gate_up_bias[:, 0::2].reshape(E, 1, I)
    gbo = gate_up_bias[:, 1::2].reshape(E, 1, I)
    db3 = down_bias.reshape(E, 1, H)
    y = _gmm(xs, ps, blk_e, gate_up_proj, gbe, gbo, down_proj, db3)

    combine = _make_combine(T, H, T // NSC)
    oa, ob = combine(pos_flat, y)
    out = _tc_add(oa, ob)
    return out.reshape(B, S, H)
